# baseline (speedup 1.0000x reference)
import jax
import jax.numpy as jnp
from jax.experimental import pallas as pl


_B = 96


def _mute_block_kernel(x_ref, o_ref):
    x = x_ref[...]
    bits = jax.lax.bitcast_convert_type(x, jnp.uint32)
    absbits = bits & jnp.uint32(0x7FFFFFFF)
    muted_bits = (bits & jnp.uint32(0x807FFFFF)) | jnp.uint32(126 << 23)
    muted = jax.lax.bitcast_convert_type(muted_bits, jnp.float32)
    shape = x.shape
    r = jax.lax.broadcasted_iota(jnp.int32, shape, 1)
    c = jax.lax.broadcasted_iota(jnp.int32, shape, 2)
    on_grid = ((r & 7) == 0) & ((c & 7) == 0)
    apply = on_grid & (absbits >= jnp.uint32(0x3F800000))
    o_ref[...] = jnp.where(apply, muted, x)


from jax.experimental.pallas import tpu as pltpu


def kernel(inputs):
    n, h, w = inputs.shape
    grid = (pl.cdiv(n, _B),)
    return pl.pallas_call(
        _mute_block_kernel,
        grid=grid,
        in_specs=[pl.BlockSpec((_B, h, w), lambda i: (i, 0, 0))],
        out_specs=pl.BlockSpec((_B, h, w), lambda i: (i, 0, 0)),
        out_shape=jax.ShapeDtypeStruct(inputs.shape, inputs.dtype),
        compiler_params=pltpu.CompilerParams(
            dimension_semantics=(pltpu.PARALLEL,),
        ),
    )(inputs)
